# baseline (speedup 1.0000x reference)
import jax
import jax.numpy as jnp
from jax import lax
from jax.experimental import pallas as pl
from jax.experimental.pallas import tpu as pltpu
from jax.experimental.pallas import tpu_sc as plsc

_N = 65536
_D = 64
_NNZ = 4194304

_NS = 16
_NUM_CHUNKS = 4
_CHUNK = _N // _NUM_CHUNKS
_B = 2048
_G = 128
_CAP = 32
_EPT = _NNZ // _NS
_NBLK = _EPT // _B
_ROWS_PER_TILE = _CHUNK // _NS
_ABL = 4


def _sc_body(spot_hbm, rows_hbm, cols_hbm, vals_hbm, out_hbm,
             rbuf, cbuf, vbuf, ccomp, vcomp, rcomp, gbuf2, zbuf, accum,
             gsem, ssem):
  cid = lax.axis_index("c")
  sid = lax.axis_index("s")
  iota = lax.iota(jnp.int32, 16)

  def _stage_issue(blk, par):
    base = sid * _EPT + blk * _B
    pltpu.async_copy(rows_hbm.at[pl.ds(base, _B)], rbuf.at[par], ssem)
    pltpu.async_copy(cols_hbm.at[pl.ds(base, _B)], cbuf.at[par], ssem)
    pltpu.async_copy(vals_hbm.at[pl.ds(base, _B)], vbuf.at[par], ssem)

  def _stage_wait(par):
    pltpu.make_async_copy(rows_hbm.at[pl.ds(0, _B)], rbuf.at[par], ssem).wait()
    pltpu.make_async_copy(cols_hbm.at[pl.ds(0, _B)], cbuf.at[par], ssem).wait()
    pltpu.make_async_copy(vals_hbm.at[pl.ds(0, _B)], vbuf.at[par], ssem).wait()

  def _gather_issue(j):
    pltpu.async_copy(spot_hbm.at[ccomp.at[j & (_CAP - 1)]],
                     gbuf2.at[j & 1], gsem.at[j & 1])

  def _gather_wait(j):
    pltpu.make_async_copy(spot_hbm.at[ccomp.at[j & (_CAP - 1)]],
                          gbuf2.at[j & 1], gsem.at[j & 1]).wait()

  def _bcast_lane(vec, l):
    idx = jnp.full((16, 1), l, jnp.int32)
    return lax.gather(
        vec, idx,
        lax.GatherDimensionNumbers(offset_dims=(), collapsed_slice_dims=(0,),
                                   start_index_map=(0,)),
        (1,), mode=lax.GatherScatterMode.PROMISE_IN_BOUNDS)

  def _mul(j):
    par = j & 1
    row = j & (_CAP - 1)

    def _q(q, c2):
      vv = vcomp[row, pl.ds(q * 16, 16)]
      e0 = q * 16
      for l in range(16):
        bc = _bcast_lane(vv, l)
        for k in range(_D // 16):
          gbuf2[par, e0 + l, pl.ds(k * 16, 16)] = (
              gbuf2[par, e0 + l, pl.ds(k * 16, 16)] * bc)
      return c2
    lax.fori_loop(0, _G // 16, _q, 0)

  def _scatter(j):
    pltpu.sync_copy(gbuf2.at[j & 1], accum.at[rcomp.at[j & (_CAP - 1)]],
                    add=True)

  def _zinit(e, carry):
    for k in range(_D // 16):
      zbuf[e, pl.ds(k * 16, 16)] = jnp.zeros((16,), jnp.float32)
    return carry
  lax.fori_loop(0, _G, _zinit, 0)

  def _idxinit(r, carry):
    for k in range(_G // 16):
      ccomp[r, pl.ds(k * 16, 16)] = jnp.zeros((16,), jnp.int32)
      rcomp[r, pl.ds(k * 16, 16)] = jnp.zeros((16,), jnp.int32)
    return carry
  lax.fori_loop(0, _CAP, _idxinit, 0)

  for p in range(_NUM_CHUNKS // 2):
    chunk = 2 * cid + p
    lo = chunk * _CHUNK

    for z in range(_ROWS_PER_TILE // _G):
      pltpu.sync_copy(zbuf, accum.at[pl.ds(sid * _ROWS_PER_TILE + z * _G, _G)])
    plsc.subcore_barrier()

    _stage_issue(0, 0)

    def _block(blk, carry):
      count0, done0 = carry
      pb = blk & 1
      _stage_wait(pb)

      @pl.when(blk + 1 < _NBLK)
      def _():
        _stage_issue(blk + 1, 1 - pb)

      def _compact(i, count):
        r = rbuf[pb, pl.ds(i * 16, 16)]
        c = cbuf[pb, pl.ds(i * 16, 16)]
        v = vbuf[pb, pl.ds(i * 16, 16)]
        rl = r - lo
        m = (rl >= 0) & (rl < _CHUNK)
        inc = jnp.cumsum(jnp.where(m, jnp.int32(1), jnp.int32(0)))
        pos = count + inc - 1
        pj = lax.bitwise_and(lax.shift_right_logical(pos, 7),
                             jnp.int32(_CAP - 1))
        pi = lax.bitwise_and(pos, jnp.int32(_G - 1))
        plsc.store_scatter(ccomp, [pj, pi], c, mask=m)
        plsc.store_scatter(vcomp, [pj, pi], v, mask=m)
        plsc.store_scatter(rcomp, [pj, pi], rl, mask=m)
        return count + inc[15]

      count1 = lax.fori_loop(0, _B // 16, _compact, count0)
      done1 = lax.shift_right_logical(count1, 7)

      if _ABL >= 1:
        @pl.when(done1 > done0)
        def _():
          _gather_issue(done0)

      def _chunkproc(j, c2):
        @pl.when(j + 1 < done1)
        def _():
          _gather_issue(j + 1)
        _gather_wait(j)
        if _ABL >= 2:
          _mul(j)
        if _ABL >= 3:
          _scatter(j)
        return c2
      if _ABL >= 1:
        lax.fori_loop(done0, done1, _chunkproc, 0)
      return (count1, done1)

    count, done = lax.fori_loop(
        0, _NBLK, _block, (jnp.int32(0), jnp.int32(0)))

    @pl.when(lax.bitwise_and(count, jnp.int32(_G - 1)) > 0)
    def _():
      def _pad(g, carry):
        row = lax.bitwise_and(lax.shift_right_logical(g, 3),
                              jnp.int32(_CAP - 1))
        col = lax.bitwise_and(g, jnp.int32(7)) * 16
        old = vcomp[row, pl.ds(col, 16)]
        keep = (g * 16 + iota) < count
        vcomp[row, pl.ds(col, 16)] = jnp.where(keep, old, jnp.float32(0.0))
        return carry
      lax.fori_loop(lax.shift_right_logical(count, 4), (done + 1) * 8,
                    _pad, 0)
      if _ABL >= 1:
        _gather_issue(done)
        _gather_wait(done)
      if _ABL >= 2:
        _mul(done)
      if _ABL >= 3:
        _scatter(done)

    plsc.subcore_barrier()

    pltpu.sync_copy(
        accum.at[pl.ds(sid * _ROWS_PER_TILE, _ROWS_PER_TILE)],
        out_hbm.at[pl.ds(lo + sid * _ROWS_PER_TILE, _ROWS_PER_TILE)])
    plsc.subcore_barrier()


_kern = pl.kernel(
    _sc_body,
    out_type=jax.ShapeDtypeStruct((_N, _D), jnp.float32),
    mesh=plsc.VectorSubcoreMesh(core_axis_name="c", subcore_axis_name="s"),
    compiler_params=pltpu.CompilerParams(
        needs_layout_passes=False, use_tc_tiling_on_sc=False),
    scratch_types=[
        pltpu.VMEM((2, _B), jnp.int32),
        pltpu.VMEM((2, _B), jnp.int32),
        pltpu.VMEM((2, _B), jnp.float32),
        pltpu.VMEM((_CAP, _G), jnp.int32),
        pltpu.VMEM((_CAP, _G), jnp.float32),
        pltpu.VMEM((_CAP, _G), jnp.int32),
        pltpu.VMEM((2, _G, _D), jnp.float32),
        pltpu.VMEM((_G, _D), jnp.float32),
        pltpu.VMEM_SHARED((_CHUNK, _D), jnp.float32),
        pltpu.SemaphoreType.DMA((2,)),
        pltpu.SemaphoreType.DMA,
    ],
)


def kernel(spot_x, A_rows, A_cols, A_vals):
  rows = A_rows.astype(jnp.int32)
  cols = A_cols.astype(jnp.int32)
  return _kern(spot_x, rows, cols, A_vals)


# speedup vs baseline: 1.7351x; 1.7351x over previous
import jax
import jax.numpy as jnp
from jax import lax
from jax.experimental import pallas as pl
from jax.experimental.pallas import tpu as pltpu
from jax.experimental.pallas import tpu_sc as plsc

_N = 65536
_D = 64
_NNZ = 4194304

_NS = 16
_NUM_CHUNKS = 4
_CHUNK = _N // _NUM_CHUNKS
_B = 2048
_G = 128
_CAP = 32
_EPT = _NNZ // _NS
_NBLK = _EPT // _B
_ROWS_PER_TILE = _CHUNK // _NS
_ABL = 4


def _sc_body(spot_hbm, rows_hbm, cols_hbm, vals_hbm, out_hbm,
             rbuf, cbuf, vbuf, ccomp, vcomp, rcomp, gbuf2, zbuf, accum,
             gsem, ssem):
  cid = lax.axis_index("c")
  sid = lax.axis_index("s")
  iota = lax.iota(jnp.int32, 16)

  def _stage_issue(blk, par):
    base = sid * _EPT + blk * _B
    pltpu.async_copy(rows_hbm.at[pl.ds(base, _B)], rbuf.at[par], ssem)
    pltpu.async_copy(cols_hbm.at[pl.ds(base, _B)], cbuf.at[par], ssem)
    pltpu.async_copy(vals_hbm.at[pl.ds(base, _B)], vbuf.at[par], ssem)

  def _stage_wait(par):
    pltpu.make_async_copy(rows_hbm.at[pl.ds(0, _B)], rbuf.at[par], ssem).wait()
    pltpu.make_async_copy(cols_hbm.at[pl.ds(0, _B)], cbuf.at[par], ssem).wait()
    pltpu.make_async_copy(vals_hbm.at[pl.ds(0, _B)], vbuf.at[par], ssem).wait()

  def _gather_issue(j):
    pltpu.async_copy(spot_hbm.at[ccomp.at[j & (_CAP - 1)]],
                     gbuf2.at[j & 1], gsem.at[j & 1])

  def _gather_wait(j):
    pltpu.make_async_copy(spot_hbm.at[ccomp.at[j & (_CAP - 1)]],
                          gbuf2.at[j & 1], gsem.at[j & 1]).wait()

  def _bcast_lane(vec, l):
    idx = jnp.full((16, 1), l, jnp.int32)
    return lax.gather(
        vec, idx,
        lax.GatherDimensionNumbers(offset_dims=(), collapsed_slice_dims=(0,),
                                   start_index_map=(0,)),
        (1,), mode=lax.GatherScatterMode.PROMISE_IN_BOUNDS)

  def _mul(j):
    par = j & 1
    row = j & (_CAP - 1)

    @plsc.parallel_loop(0, _G // 16, unroll=2)
    def _q(q):
      vv = vcomp[row, pl.ds(q * 16, 16)]
      e0 = q * 16
      for l in range(16):
        bc = _bcast_lane(vv, l)
        for k in range(_D // 16):
          gbuf2[par, e0 + l, pl.ds(k * 16, 16)] = (
              gbuf2[par, e0 + l, pl.ds(k * 16, 16)] * bc)

  def _scatter(j):
    pltpu.sync_copy(gbuf2.at[j & 1], accum.at[rcomp.at[j & (_CAP - 1)]],
                    add=True)

  def _zinit(e, carry):
    for k in range(_D // 16):
      zbuf[e, pl.ds(k * 16, 16)] = jnp.zeros((16,), jnp.float32)
    return carry
  lax.fori_loop(0, _G, _zinit, 0)

  def _idxinit(r, carry):
    for k in range(_G // 16):
      ccomp[r, pl.ds(k * 16, 16)] = jnp.zeros((16,), jnp.int32)
      rcomp[r, pl.ds(k * 16, 16)] = jnp.zeros((16,), jnp.int32)
    return carry
  lax.fori_loop(0, _CAP, _idxinit, 0)

  for p in range(_NUM_CHUNKS // 2):
    chunk = 2 * cid + p
    lo = chunk * _CHUNK

    for z in range(_ROWS_PER_TILE // _G):
      pltpu.sync_copy(zbuf, accum.at[pl.ds(sid * _ROWS_PER_TILE + z * _G, _G)])
    plsc.subcore_barrier()

    _stage_issue(0, 0)

    def _block(blk, carry):
      count0, done0 = carry
      pb = blk & 1
      _stage_wait(pb)

      @pl.when(blk + 1 < _NBLK)
      def _():
        _stage_issue(blk + 1, 1 - pb)

      def _compact(i, count):
        r = rbuf[pb, pl.ds(i * 16, 16)]
        c = cbuf[pb, pl.ds(i * 16, 16)]
        v = vbuf[pb, pl.ds(i * 16, 16)]
        rl = r - lo
        m = (rl >= 0) & (rl < _CHUNK)
        inc = jnp.cumsum(jnp.where(m, jnp.int32(1), jnp.int32(0)))
        pos = count + inc - 1
        pj = lax.bitwise_and(lax.shift_right_logical(pos, 7),
                             jnp.int32(_CAP - 1))
        pi = lax.bitwise_and(pos, jnp.int32(_G - 1))
        plsc.store_scatter(ccomp, [pj, pi], c, mask=m)
        plsc.store_scatter(vcomp, [pj, pi], v, mask=m)
        plsc.store_scatter(rcomp, [pj, pi], rl, mask=m)
        return count + inc[15]

      count1 = lax.fori_loop(0, _B // 16, _compact, count0)
      done1 = lax.shift_right_logical(count1, 7)

      if _ABL >= 1:
        @pl.when(done1 > done0)
        def _():
          _gather_issue(done0)

      def _chunkproc(j, c2):
        @pl.when(j + 1 < done1)
        def _():
          _gather_issue(j + 1)
        _gather_wait(j)
        if _ABL >= 2:
          _mul(j)
        if _ABL >= 3:
          _scatter(j)
        return c2
      if _ABL >= 1:
        lax.fori_loop(done0, done1, _chunkproc, 0)
      return (count1, done1)

    count, done = lax.fori_loop(
        0, _NBLK, _block, (jnp.int32(0), jnp.int32(0)))

    @pl.when(lax.bitwise_and(count, jnp.int32(_G - 1)) > 0)
    def _():
      def _pad(g, carry):
        row = lax.bitwise_and(lax.shift_right_logical(g, 3),
                              jnp.int32(_CAP - 1))
        col = lax.bitwise_and(g, jnp.int32(7)) * 16
        old = vcomp[row, pl.ds(col, 16)]
        keep = (g * 16 + iota) < count
        vcomp[row, pl.ds(col, 16)] = jnp.where(keep, old, jnp.float32(0.0))
        return carry
      lax.fori_loop(lax.shift_right_logical(count, 4), (done + 1) * 8,
                    _pad, 0)
      if _ABL >= 1:
        _gather_issue(done)
        _gather_wait(done)
      if _ABL >= 2:
        _mul(done)
      if _ABL >= 3:
        _scatter(done)

    plsc.subcore_barrier()

    pltpu.sync_copy(
        accum.at[pl.ds(sid * _ROWS_PER_TILE, _ROWS_PER_TILE)],
        out_hbm.at[pl.ds(lo + sid * _ROWS_PER_TILE, _ROWS_PER_TILE)])
    plsc.subcore_barrier()


_kern = pl.kernel(
    _sc_body,
    out_type=jax.ShapeDtypeStruct((_N, _D), jnp.float32),
    mesh=plsc.VectorSubcoreMesh(core_axis_name="c", subcore_axis_name="s"),
    compiler_params=pltpu.CompilerParams(
        needs_layout_passes=False, use_tc_tiling_on_sc=False),
    scratch_types=[
        pltpu.VMEM((2, _B), jnp.int32),
        pltpu.VMEM((2, _B), jnp.int32),
        pltpu.VMEM((2, _B), jnp.float32),
        pltpu.VMEM((_CAP, _G), jnp.int32),
        pltpu.VMEM((_CAP, _G), jnp.float32),
        pltpu.VMEM((_CAP, _G), jnp.int32),
        pltpu.VMEM((2, _G, _D), jnp.float32),
        pltpu.VMEM((_G, _D), jnp.float32),
        pltpu.VMEM_SHARED((_CHUNK, _D), jnp.float32),
        pltpu.SemaphoreType.DMA((2,)),
        pltpu.SemaphoreType.DMA,
    ],
)


def kernel(spot_x, A_rows, A_cols, A_vals):
  rows = A_rows.astype(jnp.int32)
  cols = A_cols.astype(jnp.int32)
  return _kern(spot_x, rows, cols, A_vals)


# speedup vs baseline: 2.0013x; 1.1534x over previous
import jax
import jax.numpy as jnp
from jax import lax
from jax.experimental import pallas as pl
from jax.experimental.pallas import tpu as pltpu
from jax.experimental.pallas import tpu_sc as plsc

_N = 65536
_D = 64
_NNZ = 4194304

_NS = 16
_NUM_CHUNKS = 4
_CHUNK = _N // _NUM_CHUNKS
_B = 2048
_G = 128
_CAP = 32
_NSLOT = 4
_EPT = _NNZ // _NS
_NBLK = _EPT // _B
_ROWS_PER_TILE = _CHUNK // _NS


def _sc_body(spot_hbm, rows_hbm, cols_hbm, vals_hbm, out_hbm,
             rbuf, cbuf, vbuf, ccomp, vcomp, rcomp, gbuf, accum,
             gsem, csem, ssem):
  cid = lax.axis_index("c")
  sid = lax.axis_index("s")
  iota = lax.iota(jnp.int32, 16)

  def _stage_issue(blk, par):
    base = sid * _EPT + blk * _B
    pltpu.async_copy(rows_hbm.at[pl.ds(base, _B)], rbuf.at[par], ssem)
    pltpu.async_copy(cols_hbm.at[pl.ds(base, _B)], cbuf.at[par], ssem)
    pltpu.async_copy(vals_hbm.at[pl.ds(base, _B)], vbuf.at[par], ssem)

  def _stage_wait(par):
    pltpu.make_async_copy(rows_hbm.at[pl.ds(0, _B)], rbuf.at[par], ssem).wait()
    pltpu.make_async_copy(cols_hbm.at[pl.ds(0, _B)], cbuf.at[par], ssem).wait()
    pltpu.make_async_copy(vals_hbm.at[pl.ds(0, _B)], vbuf.at[par], ssem).wait()

  def _gather_issue(j):
    @pl.when(j >= _NSLOT)
    def _():
      pltpu.make_async_copy(
          gbuf.at[j & (_NSLOT - 1)],
          accum.at[rcomp.at[(j - _NSLOT) & (_CAP - 1)]],
          csem.at[j & (_NSLOT - 1)]).wait()
    pltpu.async_copy(spot_hbm.at[ccomp.at[j & (_CAP - 1)]],
                     gbuf.at[j & (_NSLOT - 1)], gsem.at[j & (_NSLOT - 1)])

  def _gather_wait(j):
    pltpu.make_async_copy(spot_hbm.at[ccomp.at[j & (_CAP - 1)]],
                          gbuf.at[j & (_NSLOT - 1)],
                          gsem.at[j & (_NSLOT - 1)]).wait()

  def _scatter_issue(j):
    pltpu.async_copy(gbuf.at[j & (_NSLOT - 1)],
                     accum.at[rcomp.at[j & (_CAP - 1)]],
                     csem.at[j & (_NSLOT - 1)], add=True)

  def _scatter_drain(t):
    pltpu.make_async_copy(gbuf.at[t & (_NSLOT - 1)],
                          accum.at[rcomp.at[t & (_CAP - 1)]],
                          csem.at[t & (_NSLOT - 1)]).wait()

  def _bcast_lane(vec, l):
    idx = jnp.full((16, 1), l, jnp.int32)
    return lax.gather(
        vec, idx,
        lax.GatherDimensionNumbers(offset_dims=(), collapsed_slice_dims=(0,),
                                   start_index_map=(0,)),
        (1,), mode=lax.GatherScatterMode.PROMISE_IN_BOUNDS)

  def _mul(j):
    par = j & (_NSLOT - 1)
    row = j & (_CAP - 1)

    @plsc.parallel_loop(0, _G // 16, unroll=2)
    def _q(q):
      vv = vcomp[row, pl.ds(q * 16, 16)]
      e0 = q * 16
      for l in range(16):
        bc = _bcast_lane(vv, l)
        for k in range(_D // 16):
          gbuf[par, e0 + l, pl.ds(k * 16, 16)] = (
              gbuf[par, e0 + l, pl.ds(k * 16, 16)] * bc)

  def _idxinit(r, carry):
    for k in range(_G // 16):
      ccomp[r, pl.ds(k * 16, 16)] = jnp.zeros((16,), jnp.int32)
      rcomp[r, pl.ds(k * 16, 16)] = jnp.zeros((16,), jnp.int32)
    return carry
  lax.fori_loop(0, _CAP, _idxinit, 0)

  for p in range(_NUM_CHUNKS // 2):
    chunk = 2 * cid + p
    lo = chunk * _CHUNK

    def _zinit(e, carry):
      for k in range(_D // 16):
        gbuf[0, e, pl.ds(k * 16, 16)] = jnp.zeros((16,), jnp.float32)
      return carry
    lax.fori_loop(0, _G, _zinit, 0)
    for z in range(_ROWS_PER_TILE // _G):
      pltpu.sync_copy(gbuf.at[0],
                      accum.at[pl.ds(sid * _ROWS_PER_TILE + z * _G, _G)])
    plsc.subcore_barrier()

    _stage_issue(0, 0)

    def _block(blk, carry):
      cvec0, done0 = carry
      pb = blk & 1
      _stage_wait(pb)

      @pl.when(blk + 1 < _NBLK)
      def _():
        _stage_issue(blk + 1, 1 - pb)

      def _compact(i, cvec):
        r = rbuf[pb, pl.ds(i * 16, 16)]
        c = cbuf[pb, pl.ds(i * 16, 16)]
        v = vbuf[pb, pl.ds(i * 16, 16)]
        rl = r - lo
        m = (rl >= 0) & (rl < _CHUNK)
        inc = jnp.cumsum(jnp.where(m, jnp.int32(1), jnp.int32(0)))
        pos = cvec + inc - 1
        pj = lax.bitwise_and(lax.shift_right_logical(pos, 7),
                             jnp.int32(_CAP - 1))
        pi = lax.bitwise_and(pos, jnp.int32(_G - 1))
        plsc.store_scatter(ccomp, [pj, pi], c, mask=m)
        plsc.store_scatter(vcomp, [pj, pi], v, mask=m)
        plsc.store_scatter(rcomp, [pj, pi], rl, mask=m)
        return _bcast_lane(cvec + inc, 15)

      cvec1 = lax.fori_loop(0, _B // 16, _compact, cvec0)
      count1 = cvec1[0]
      done1 = lax.shift_right_logical(count1, 7)

      @pl.when(done1 > done0)
      def _():
        _gather_issue(done0)

      def _chunkproc(j, c2):
        @pl.when(j + 1 < done1)
        def _():
          _gather_issue(j + 1)
        _gather_wait(j)
        _mul(j)
        _scatter_issue(j)
        return c2
      lax.fori_loop(done0, done1, _chunkproc, 0)
      return (cvec1, done1)

    cvec, done = lax.fori_loop(
        0, _NBLK, _block, (jnp.zeros((16,), jnp.int32), jnp.int32(0)))
    count = cvec[0]
    rem = lax.bitwise_and(count, jnp.int32(_G - 1))

    @pl.when(rem > 0)
    def _():
      def _pad(g, carry):
        row = lax.bitwise_and(lax.shift_right_logical(g, 3),
                              jnp.int32(_CAP - 1))
        col = lax.bitwise_and(g, jnp.int32(7)) * 16
        old = vcomp[row, pl.ds(col, 16)]
        keep = (g * 16 + iota) < count
        vcomp[row, pl.ds(col, 16)] = jnp.where(keep, old, jnp.float32(0.0))
        return carry
      lax.fori_loop(lax.shift_right_logical(count, 4), (done + 1) * 8,
                    _pad, 0)
      _gather_issue(done)
      _gather_wait(done)
      _mul(done)
      _scatter_issue(done)

    total = done + jnp.where(rem > 0, jnp.int32(1), jnp.int32(0))

    def _fin(t, c2):
      _scatter_drain(t)
      return c2
    lax.fori_loop(jnp.maximum(total - _NSLOT, 0), total, _fin, 0)

    plsc.subcore_barrier()

    pltpu.sync_copy(
        accum.at[pl.ds(sid * _ROWS_PER_TILE, _ROWS_PER_TILE)],
        out_hbm.at[pl.ds(lo + sid * _ROWS_PER_TILE, _ROWS_PER_TILE)])
    plsc.subcore_barrier()


_kern = pl.kernel(
    _sc_body,
    out_type=jax.ShapeDtypeStruct((_N, _D), jnp.float32),
    mesh=plsc.VectorSubcoreMesh(core_axis_name="c", subcore_axis_name="s"),
    compiler_params=pltpu.CompilerParams(
        needs_layout_passes=False, use_tc_tiling_on_sc=False),
    scratch_types=[
        pltpu.VMEM((2, _B), jnp.int32),
        pltpu.VMEM((2, _B), jnp.int32),
        pltpu.VMEM((2, _B), jnp.float32),
        pltpu.VMEM((_CAP, _G), jnp.int32),
        pltpu.VMEM((_CAP, _G), jnp.float32),
        pltpu.VMEM((_CAP, _G), jnp.int32),
        pltpu.VMEM((_NSLOT, _G, _D), jnp.float32),
        pltpu.VMEM_SHARED((_CHUNK, _D), jnp.float32),
        pltpu.SemaphoreType.DMA((_NSLOT,)),
        pltpu.SemaphoreType.DMA((_NSLOT,)),
        pltpu.SemaphoreType.DMA,
    ],
)


def kernel(spot_x, A_rows, A_cols, A_vals):
  rows = A_rows.astype(jnp.int32)
  cols = A_cols.astype(jnp.int32)
  return _kern(spot_x, rows, cols, A_vals)


# speedup vs baseline: 2.5707x; 1.2845x over previous
import jax
import jax.numpy as jnp
from jax import lax
from jax.experimental import pallas as pl
from jax.experimental.pallas import tpu as pltpu
from jax.experimental.pallas import tpu_sc as plsc

_N = 65536
_D = 64
_NNZ = 4194304

_NS = 16
_NUM_CHUNKS = 4
_CHUNK = _N // _NUM_CHUNKS
_B = 2048
_G = 128
_CAP = 32
_NSLOT = 4
_EPT = _NNZ // _NS
_NBLK = _EPT // _B
_ROWS_PER_TILE = _CHUNK // _NS


def _sc_body(spot_hbm, rows_hbm, cols_hbm, vals_hbm, out_hbm,
             rbuf, cbuf, vbuf, ccomp, vcomp, rcomp, gbuf, accum,
             gsem, csem, ssem):
  cid = lax.axis_index("c")
  sid = lax.axis_index("s")
  iota = lax.iota(jnp.int32, 16)

  def _stage_issue(blk, par):
    base = sid * _EPT + blk * _B
    pltpu.async_copy(rows_hbm.at[pl.ds(base, _B)], rbuf.at[par], ssem)
    pltpu.async_copy(cols_hbm.at[pl.ds(base, _B)], cbuf.at[par], ssem)
    pltpu.async_copy(vals_hbm.at[pl.ds(base, _B)], vbuf.at[par], ssem)

  def _stage_wait(par):
    pltpu.make_async_copy(rows_hbm.at[pl.ds(0, _B)], rbuf.at[par], ssem).wait()
    pltpu.make_async_copy(cols_hbm.at[pl.ds(0, _B)], cbuf.at[par], ssem).wait()
    pltpu.make_async_copy(vals_hbm.at[pl.ds(0, _B)], vbuf.at[par], ssem).wait()

  def _gather_issue(j):
    @pl.when(j >= _NSLOT)
    def _():
      pltpu.make_async_copy(
          gbuf.at[j & (_NSLOT - 1)],
          accum.at[rcomp.at[(j - _NSLOT) & (_CAP - 1)]],
          csem.at[j & (_NSLOT - 1)]).wait()
    pltpu.async_copy(spot_hbm.at[ccomp.at[j & (_CAP - 1)]],
                     gbuf.at[j & (_NSLOT - 1)], gsem.at[j & (_NSLOT - 1)])

  def _gather_wait(j):
    pltpu.make_async_copy(spot_hbm.at[ccomp.at[j & (_CAP - 1)]],
                          gbuf.at[j & (_NSLOT - 1)],
                          gsem.at[j & (_NSLOT - 1)]).wait()

  def _scatter_issue(j):
    pltpu.async_copy(gbuf.at[j & (_NSLOT - 1)],
                     accum.at[rcomp.at[j & (_CAP - 1)]],
                     csem.at[j & (_NSLOT - 1)], add=True)

  def _scatter_drain(t):
    pltpu.make_async_copy(gbuf.at[t & (_NSLOT - 1)],
                          accum.at[rcomp.at[t & (_CAP - 1)]],
                          csem.at[t & (_NSLOT - 1)]).wait()

  def _bcast_lane(vec, l):
    idx = jnp.full((16, 1), l, jnp.int32)
    return lax.gather(
        vec, idx,
        lax.GatherDimensionNumbers(offset_dims=(), collapsed_slice_dims=(0,),
                                   start_index_map=(0,)),
        (1,), mode=lax.GatherScatterMode.PROMISE_IN_BOUNDS)

  def _mul(j):
    par = j & (_NSLOT - 1)
    row = j & (_CAP - 1)

    @plsc.parallel_loop(0, _G // 16, unroll=2)
    def _q(q):
      vv = vcomp[row, pl.ds(q * 16, 16)]
      e0 = q * 16
      for l in range(16):
        bc = _bcast_lane(vv, l)
        for k in range(_D // 16):
          gbuf[par, e0 + l, pl.ds(k * 16, 16)] = (
              gbuf[par, e0 + l, pl.ds(k * 16, 16)] * bc)

  def _idxinit(r, carry):
    for k in range(_G // 16):
      ccomp[r, pl.ds(k * 16, 16)] = jnp.zeros((16,), jnp.int32)
      rcomp[r, pl.ds(k * 16, 16)] = jnp.zeros((16,), jnp.int32)
    return carry
  lax.fori_loop(0, _CAP, _idxinit, 0)

  for p in range(_NUM_CHUNKS // 2):
    chunk = 2 * cid + p
    lo = chunk * _CHUNK

    def _zinit(e, carry):
      for k in range(_D // 16):
        gbuf[0, e, pl.ds(k * 16, 16)] = jnp.zeros((16,), jnp.float32)
      return carry
    lax.fori_loop(0, _G, _zinit, 0)
    for z in range(_ROWS_PER_TILE // _G):
      pltpu.sync_copy(gbuf.at[0],
                      accum.at[pl.ds(sid * _ROWS_PER_TILE + z * _G, _G)])
    plsc.subcore_barrier()

    _stage_issue(0, 0)

    def _block(blk, carry):
      cvec0, done0 = carry
      pb = blk & 1
      _stage_wait(pb)

      @pl.when(blk + 1 < _NBLK)
      def _():
        _stage_issue(blk + 1, 1 - pb)

      def _compact(i, cvec):
        r = rbuf[pb, pl.ds(i * 16, 16)]
        c = cbuf[pb, pl.ds(i * 16, 16)]
        v = vbuf[pb, pl.ds(i * 16, 16)]
        rl = r - lo
        m = (rl >= 0) & (rl < _CHUNK)
        inc = jnp.cumsum(jnp.where(m, jnp.int32(1), jnp.int32(0)))
        pos = cvec + inc - 1
        pj = lax.bitwise_and(lax.shift_right_logical(pos, 7),
                             jnp.int32(_CAP - 1))
        pi = lax.bitwise_and(pos, jnp.int32(_G - 1))
        plsc.store_scatter(ccomp, [pj, pi], c, mask=m)
        plsc.store_scatter(vcomp, [pj, pi], v, mask=m)
        plsc.store_scatter(rcomp, [pj, pi], rl, mask=m)
        return _bcast_lane(cvec + inc, 15)

      cvec1 = plsc.parallel_loop(0, _B // 16, unroll=2, carry=cvec0)(_compact)
      count1 = cvec1[0]
      done1 = lax.shift_right_logical(count1, 7)

      @pl.when(done1 > done0)
      def _():
        _gather_issue(done0)

      def _chunkproc(j, c2):
        @pl.when(j + 1 < done1)
        def _():
          _gather_issue(j + 1)
        _gather_wait(j)
        _mul(j)
        _scatter_issue(j)
        return c2
      lax.fori_loop(done0, done1, _chunkproc, 0)
      return (cvec1, done1)

    cvec, done = lax.fori_loop(
        0, _NBLK, _block, (jnp.zeros((16,), jnp.int32), jnp.int32(0)))
    count = cvec[0]
    rem = lax.bitwise_and(count, jnp.int32(_G - 1))

    @pl.when(rem > 0)
    def _():
      def _pad(g, carry):
        row = lax.bitwise_and(lax.shift_right_logical(g, 3),
                              jnp.int32(_CAP - 1))
        col = lax.bitwise_and(g, jnp.int32(7)) * 16
        old = vcomp[row, pl.ds(col, 16)]
        keep = (g * 16 + iota) < count
        vcomp[row, pl.ds(col, 16)] = jnp.where(keep, old, jnp.float32(0.0))
        return carry
      lax.fori_loop(lax.shift_right_logical(count, 4), (done + 1) * 8,
                    _pad, 0)
      _gather_issue(done)
      _gather_wait(done)
      _mul(done)
      _scatter_issue(done)

    total = done + jnp.where(rem > 0, jnp.int32(1), jnp.int32(0))

    def _fin(t, c2):
      _scatter_drain(t)
      return c2
    lax.fori_loop(jnp.maximum(total - _NSLOT, 0), total, _fin, 0)

    plsc.subcore_barrier()

    pltpu.sync_copy(
        accum.at[pl.ds(sid * _ROWS_PER_TILE, _ROWS_PER_TILE)],
        out_hbm.at[pl.ds(lo + sid * _ROWS_PER_TILE, _ROWS_PER_TILE)])
    plsc.subcore_barrier()


_kern = pl.kernel(
    _sc_body,
    out_type=jax.ShapeDtypeStruct((_N, _D), jnp.float32),
    mesh=plsc.VectorSubcoreMesh(core_axis_name="c", subcore_axis_name="s"),
    compiler_params=pltpu.CompilerParams(
        needs_layout_passes=False, use_tc_tiling_on_sc=False),
    scratch_types=[
        pltpu.VMEM((2, _B), jnp.int32),
        pltpu.VMEM((2, _B), jnp.int32),
        pltpu.VMEM((2, _B), jnp.float32),
        pltpu.VMEM((_CAP, _G), jnp.int32),
        pltpu.VMEM((_CAP, _G), jnp.float32),
        pltpu.VMEM((_CAP, _G), jnp.int32),
        pltpu.VMEM((_NSLOT, _G, _D), jnp.float32),
        pltpu.VMEM_SHARED((_CHUNK, _D), jnp.float32),
        pltpu.SemaphoreType.DMA((_NSLOT,)),
        pltpu.SemaphoreType.DMA((_NSLOT,)),
        pltpu.SemaphoreType.DMA,
    ],
)


def kernel(spot_x, A_rows, A_cols, A_vals):
  rows = A_rows.astype(jnp.int32)
  cols = A_cols.astype(jnp.int32)
  return _kern(spot_x, rows, cols, A_vals)


# speedup vs baseline: 2.6002x; 1.0115x over previous
import jax
import jax.numpy as jnp
from jax import lax
from jax.experimental import pallas as pl
from jax.experimental.pallas import tpu as pltpu
from jax.experimental.pallas import tpu_sc as plsc

_N = 65536
_D = 64
_NNZ = 4194304

_NS = 16
_NUM_CHUNKS = 4
_CHUNK = _N // _NUM_CHUNKS
_B = 2048
_G = 128
_CAP = 32
_NSLOT = 4
_EPT = _NNZ // _NS
_NBLK = _EPT // _B
_ROWS_PER_TILE = _CHUNK // _NS


def _sc_body(spot_hbm, rows_hbm, cols_hbm, vals_hbm, out_hbm,
             rbuf, cbuf, vbuf, ccomp, vcomp, rcomp, gbuf, accum,
             gsem, csem, ssem):
  cid = lax.axis_index("c")
  sid = lax.axis_index("s")
  iota = lax.iota(jnp.int32, 16)

  def _stage_issue(blk, par):
    base = sid * _EPT + blk * _B
    pltpu.async_copy(rows_hbm.at[pl.ds(base, _B)], rbuf.at[par], ssem)
    pltpu.async_copy(cols_hbm.at[pl.ds(base, _B)], cbuf.at[par], ssem)
    pltpu.async_copy(vals_hbm.at[pl.ds(base, _B)], vbuf.at[par], ssem)

  def _stage_wait(par):
    pltpu.make_async_copy(rows_hbm.at[pl.ds(0, _B)], rbuf.at[par], ssem).wait()
    pltpu.make_async_copy(cols_hbm.at[pl.ds(0, _B)], cbuf.at[par], ssem).wait()
    pltpu.make_async_copy(vals_hbm.at[pl.ds(0, _B)], vbuf.at[par], ssem).wait()

  def _gather_issue(j):
    @pl.when(j >= _NSLOT)
    def _():
      pltpu.make_async_copy(
          gbuf.at[j & (_NSLOT - 1)],
          accum.at[rcomp.at[(j - _NSLOT) & (_CAP - 1)]],
          csem.at[j & (_NSLOT - 1)]).wait()
    pltpu.async_copy(spot_hbm.at[ccomp.at[j & (_CAP - 1)]],
                     gbuf.at[j & (_NSLOT - 1)], gsem.at[j & (_NSLOT - 1)])

  def _gather_wait(j):
    pltpu.make_async_copy(spot_hbm.at[ccomp.at[j & (_CAP - 1)]],
                          gbuf.at[j & (_NSLOT - 1)],
                          gsem.at[j & (_NSLOT - 1)]).wait()

  def _scatter_issue(j):
    pltpu.async_copy(gbuf.at[j & (_NSLOT - 1)],
                     accum.at[rcomp.at[j & (_CAP - 1)]],
                     csem.at[j & (_NSLOT - 1)], add=True)

  def _scatter_drain(t):
    pltpu.make_async_copy(gbuf.at[t & (_NSLOT - 1)],
                          accum.at[rcomp.at[t & (_CAP - 1)]],
                          csem.at[t & (_NSLOT - 1)]).wait()

  def _bcast_lane(vec, l):
    idx = jnp.full((16, 1), l, jnp.int32)
    return lax.gather(
        vec, idx,
        lax.GatherDimensionNumbers(offset_dims=(), collapsed_slice_dims=(0,),
                                   start_index_map=(0,)),
        (1,), mode=lax.GatherScatterMode.PROMISE_IN_BOUNDS)

  def _mul(j):
    par = j & (_NSLOT - 1)
    row = j & (_CAP - 1)

    @plsc.parallel_loop(0, _G // 16, unroll=4)
    def _q(q):
      vv = vcomp[row, pl.ds(q * 16, 16)]
      e0 = q * 16
      for l in range(16):
        bc = _bcast_lane(vv, l)
        for k in range(_D // 16):
          gbuf[par, e0 + l, pl.ds(k * 16, 16)] = (
              gbuf[par, e0 + l, pl.ds(k * 16, 16)] * bc)

  def _idxinit(r, carry):
    for k in range(_G // 16):
      ccomp[r, pl.ds(k * 16, 16)] = jnp.zeros((16,), jnp.int32)
      rcomp[r, pl.ds(k * 16, 16)] = jnp.zeros((16,), jnp.int32)
    return carry
  lax.fori_loop(0, _CAP, _idxinit, 0)

  for p in range(_NUM_CHUNKS // 2):
    chunk = 2 * cid + p
    lo = chunk * _CHUNK

    def _zinit(e, carry):
      for k in range(_D // 16):
        gbuf[0, e, pl.ds(k * 16, 16)] = jnp.zeros((16,), jnp.float32)
      return carry
    lax.fori_loop(0, _G, _zinit, 0)
    for z in range(_ROWS_PER_TILE // _G):
      pltpu.sync_copy(gbuf.at[0],
                      accum.at[pl.ds(sid * _ROWS_PER_TILE + z * _G, _G)])
    plsc.subcore_barrier()

    _stage_issue(0, 0)

    def _block(blk, carry):
      cvec0, done0 = carry
      pb = blk & 1
      _stage_wait(pb)

      @pl.when(blk + 1 < _NBLK)
      def _():
        _stage_issue(blk + 1, 1 - pb)

      def _compact(i, cvec):
        r = rbuf[pb, pl.ds(i * 16, 16)]
        c = cbuf[pb, pl.ds(i * 16, 16)]
        v = vbuf[pb, pl.ds(i * 16, 16)]
        rl = r - lo
        m = (rl >= 0) & (rl < _CHUNK)
        inc = jnp.cumsum(jnp.where(m, jnp.int32(1), jnp.int32(0)))
        pos = cvec + inc - 1
        pj = lax.bitwise_and(lax.shift_right_logical(pos, 7),
                             jnp.int32(_CAP - 1))
        pi = lax.bitwise_and(pos, jnp.int32(_G - 1))
        plsc.store_scatter(ccomp, [pj, pi], c, mask=m)
        plsc.store_scatter(vcomp, [pj, pi], v, mask=m)
        plsc.store_scatter(rcomp, [pj, pi], rl, mask=m)
        return _bcast_lane(cvec + inc, 15)

      cvec1 = plsc.parallel_loop(0, _B // 16, unroll=4, carry=cvec0)(_compact)
      count1 = cvec1[0]
      done1 = lax.shift_right_logical(count1, 7)

      @pl.when(done1 > done0)
      def _():
        _gather_issue(done0)

      def _chunkproc(j, c2):
        @pl.when(j + 1 < done1)
        def _():
          _gather_issue(j + 1)
        _gather_wait(j)
        _mul(j)
        _scatter_issue(j)
        return c2
      lax.fori_loop(done0, done1, _chunkproc, 0)
      return (cvec1, done1)

    cvec, done = lax.fori_loop(
        0, _NBLK, _block, (jnp.zeros((16,), jnp.int32), jnp.int32(0)))
    count = cvec[0]
    rem = lax.bitwise_and(count, jnp.int32(_G - 1))

    @pl.when(rem > 0)
    def _():
      def _pad(g, carry):
        row = lax.bitwise_and(lax.shift_right_logical(g, 3),
                              jnp.int32(_CAP - 1))
        col = lax.bitwise_and(g, jnp.int32(7)) * 16
        old = vcomp[row, pl.ds(col, 16)]
        keep = (g * 16 + iota) < count
        vcomp[row, pl.ds(col, 16)] = jnp.where(keep, old, jnp.float32(0.0))
        return carry
      lax.fori_loop(lax.shift_right_logical(count, 4), (done + 1) * 8,
                    _pad, 0)
      _gather_issue(done)
      _gather_wait(done)
      _mul(done)
      _scatter_issue(done)

    total = done + jnp.where(rem > 0, jnp.int32(1), jnp.int32(0))

    def _fin(t, c2):
      _scatter_drain(t)
      return c2
    lax.fori_loop(jnp.maximum(total - _NSLOT, 0), total, _fin, 0)

    plsc.subcore_barrier()

    pltpu.sync_copy(
        accum.at[pl.ds(sid * _ROWS_PER_TILE, _ROWS_PER_TILE)],
        out_hbm.at[pl.ds(lo + sid * _ROWS_PER_TILE, _ROWS_PER_TILE)])
    plsc.subcore_barrier()


_kern = pl.kernel(
    _sc_body,
    out_type=jax.ShapeDtypeStruct((_N, _D), jnp.float32),
    mesh=plsc.VectorSubcoreMesh(core_axis_name="c", subcore_axis_name="s"),
    compiler_params=pltpu.CompilerParams(
        needs_layout_passes=False, use_tc_tiling_on_sc=False),
    scratch_types=[
        pltpu.VMEM((2, _B), jnp.int32),
        pltpu.VMEM((2, _B), jnp.int32),
        pltpu.VMEM((2, _B), jnp.float32),
        pltpu.VMEM((_CAP, _G), jnp.int32),
        pltpu.VMEM((_CAP, _G), jnp.float32),
        pltpu.VMEM((_CAP, _G), jnp.int32),
        pltpu.VMEM((_NSLOT, _G, _D), jnp.float32),
        pltpu.VMEM_SHARED((_CHUNK, _D), jnp.float32),
        pltpu.SemaphoreType.DMA((_NSLOT,)),
        pltpu.SemaphoreType.DMA((_NSLOT,)),
        pltpu.SemaphoreType.DMA,
    ],
)


def kernel(spot_x, A_rows, A_cols, A_vals):
  rows = A_rows.astype(jnp.int32)
  cols = A_cols.astype(jnp.int32)
  return _kern(spot_x, rows, cols, A_vals)


# speedup vs baseline: 2.6235x; 1.0090x over previous
import jax
import jax.numpy as jnp
from jax import lax
from jax.experimental import pallas as pl
from jax.experimental.pallas import tpu as pltpu
from jax.experimental.pallas import tpu_sc as plsc

_N = 65536
_D = 64
_NNZ = 4194304

_NS = 16
_NUM_CHUNKS = 4
_CHUNK = _N // _NUM_CHUNKS
_B = 2048
_G = 128
_CAP = 32
_NSLOT = 4
_EPT = _NNZ // _NS
_NBLK = _EPT // _B
_ROWS_PER_TILE = _CHUNK // _NS


def _sc_body(spot_hbm, rows_hbm, cols_hbm, vals_hbm, out_hbm,
             rbuf, cbuf, vbuf, ccomp, vcomp, rcomp, gbuf, accum,
             gsem, csem, ssem):
  cid = lax.axis_index("c")
  sid = lax.axis_index("s")
  iota = lax.iota(jnp.int32, 16)

  def _stage_issue(blk, par):
    base = sid * _EPT + blk * _B
    pltpu.async_copy(rows_hbm.at[pl.ds(base, _B)], rbuf.at[par], ssem)
    pltpu.async_copy(cols_hbm.at[pl.ds(base, _B)], cbuf.at[par], ssem)
    pltpu.async_copy(vals_hbm.at[pl.ds(base, _B)], vbuf.at[par], ssem)

  def _stage_wait(par):
    pltpu.make_async_copy(rows_hbm.at[pl.ds(0, _B)], rbuf.at[par], ssem).wait()
    pltpu.make_async_copy(cols_hbm.at[pl.ds(0, _B)], cbuf.at[par], ssem).wait()
    pltpu.make_async_copy(vals_hbm.at[pl.ds(0, _B)], vbuf.at[par], ssem).wait()

  def _gather_issue(j):
    @pl.when(j >= _NSLOT)
    def _():
      pltpu.make_async_copy(
          gbuf.at[j & (_NSLOT - 1)],
          accum.at[rcomp.at[(j - _NSLOT) & (_CAP - 1)]],
          csem.at[j & (_NSLOT - 1)]).wait()
    pltpu.async_copy(spot_hbm.at[ccomp.at[j & (_CAP - 1)]],
                     gbuf.at[j & (_NSLOT - 1)], gsem.at[j & (_NSLOT - 1)])

  def _gather_wait(j):
    pltpu.make_async_copy(spot_hbm.at[ccomp.at[j & (_CAP - 1)]],
                          gbuf.at[j & (_NSLOT - 1)],
                          gsem.at[j & (_NSLOT - 1)]).wait()

  def _scatter_issue(j):
    pltpu.async_copy(gbuf.at[j & (_NSLOT - 1)],
                     accum.at[rcomp.at[j & (_CAP - 1)]],
                     csem.at[j & (_NSLOT - 1)], add=True)

  def _scatter_drain(t):
    pltpu.make_async_copy(gbuf.at[t & (_NSLOT - 1)],
                          accum.at[rcomp.at[t & (_CAP - 1)]],
                          csem.at[t & (_NSLOT - 1)]).wait()

  def _bcast_lane(vec, l):
    idx = jnp.full((16, 1), l, jnp.int32)
    return lax.gather(
        vec, idx,
        lax.GatherDimensionNumbers(offset_dims=(), collapsed_slice_dims=(0,),
                                   start_index_map=(0,)),
        (1,), mode=lax.GatherScatterMode.PROMISE_IN_BOUNDS)

  def _mul(j):
    par = j & (_NSLOT - 1)
    row = j & (_CAP - 1)

    @plsc.parallel_loop(0, _G // 16, unroll=4)
    def _q(q):
      vv = vcomp[row, pl.ds(q * 16, 16)]
      e0 = q * 16
      for l in range(16):
        bc = _bcast_lane(vv, l)
        for k in range(_D // 16):
          gbuf[par, e0 + l, pl.ds(k * 16, 16)] = (
              gbuf[par, e0 + l, pl.ds(k * 16, 16)] * bc)

  def _idxinit(r, carry):
    for k in range(_G // 16):
      ccomp[r, pl.ds(k * 16, 16)] = jnp.zeros((16,), jnp.int32)
      rcomp[r, pl.ds(k * 16, 16)] = jnp.zeros((16,), jnp.int32)
    return carry
  lax.fori_loop(0, _CAP, _idxinit, 0)

  for p in range(_NUM_CHUNKS // 2):
    chunk = 2 * cid + p
    lo = chunk * _CHUNK

    def _zinit(e, carry):
      for k in range(_D // 16):
        gbuf[0, e, pl.ds(k * 16, 16)] = jnp.zeros((16,), jnp.float32)
      return carry
    lax.fori_loop(0, _G, _zinit, 0)
    for z in range(_ROWS_PER_TILE // _G):
      pltpu.sync_copy(gbuf.at[0],
                      accum.at[pl.ds(sid * _ROWS_PER_TILE + z * _G, _G)])
    plsc.subcore_barrier()

    _stage_issue(0, 0)

    def _block(blk, carry):
      cvec0, done0 = carry
      pb = blk & 1
      _stage_wait(pb)

      @pl.when(blk + 1 < _NBLK)
      def _():
        _stage_issue(blk + 1, 1 - pb)

      def _compact(i, cvec):
        r = rbuf[pb, pl.ds(i * 16, 16)]
        c = cbuf[pb, pl.ds(i * 16, 16)]
        v = vbuf[pb, pl.ds(i * 16, 16)]
        rl = r - lo
        m = (rl >= 0) & (rl < _CHUNK)
        inc = jnp.cumsum(jnp.where(m, jnp.int32(1), jnp.int32(0)))
        pos = cvec + inc - 1
        pj = lax.bitwise_and(lax.shift_right_logical(pos, 7),
                             jnp.int32(_CAP - 1))
        pi = lax.bitwise_and(pos, jnp.int32(_G - 1))
        plsc.store_scatter(ccomp, [pj, pi], c, mask=m)
        plsc.store_scatter(vcomp, [pj, pi], v, mask=m)
        plsc.store_scatter(rcomp, [pj, pi], rl, mask=m)
        return _bcast_lane(cvec + inc, 15)

      cvec1 = plsc.parallel_loop(0, _B // 16, unroll=4, carry=cvec0)(_compact)
      count1 = cvec1[0]
      done1 = lax.shift_right_logical(count1, 7)

      @pl.when(done1 > done0)
      def _():
        _gather_issue(done0)

      @pl.when(done1 > done0 + 1)
      def _():
        _gather_issue(done0 + 1)

      def _chunkproc(j, c2):
        @pl.when(j + 2 < done1)
        def _():
          _gather_issue(j + 2)
        _gather_wait(j)
        _mul(j)
        _scatter_issue(j)
        return c2
      lax.fori_loop(done0, done1, _chunkproc, 0)
      return (cvec1, done1)

    cvec, done = lax.fori_loop(
        0, _NBLK, _block, (jnp.zeros((16,), jnp.int32), jnp.int32(0)))
    count = cvec[0]
    rem = lax.bitwise_and(count, jnp.int32(_G - 1))

    @pl.when(rem > 0)
    def _():
      def _pad(g, carry):
        row = lax.bitwise_and(lax.shift_right_logical(g, 3),
                              jnp.int32(_CAP - 1))
        col = lax.bitwise_and(g, jnp.int32(7)) * 16
        old = vcomp[row, pl.ds(col, 16)]
        keep = (g * 16 + iota) < count
        vcomp[row, pl.ds(col, 16)] = jnp.where(keep, old, jnp.float32(0.0))
        return carry
      lax.fori_loop(lax.shift_right_logical(count, 4), (done + 1) * 8,
                    _pad, 0)
      _gather_issue(done)
      _gather_wait(done)
      _mul(done)
      _scatter_issue(done)

    total = done + jnp.where(rem > 0, jnp.int32(1), jnp.int32(0))

    def _fin(t, c2):
      _scatter_drain(t)
      return c2
    lax.fori_loop(jnp.maximum(total - _NSLOT, 0), total, _fin, 0)

    plsc.subcore_barrier()

    pltpu.sync_copy(
        accum.at[pl.ds(sid * _ROWS_PER_TILE, _ROWS_PER_TILE)],
        out_hbm.at[pl.ds(lo + sid * _ROWS_PER_TILE, _ROWS_PER_TILE)])
    plsc.subcore_barrier()


_kern = pl.kernel(
    _sc_body,
    out_type=jax.ShapeDtypeStruct((_N, _D), jnp.float32),
    mesh=plsc.VectorSubcoreMesh(core_axis_name="c", subcore_axis_name="s"),
    compiler_params=pltpu.CompilerParams(
        needs_layout_passes=False, use_tc_tiling_on_sc=False),
    scratch_types=[
        pltpu.VMEM((2, _B), jnp.int32),
        pltpu.VMEM((2, _B), jnp.int32),
        pltpu.VMEM((2, _B), jnp.float32),
        pltpu.VMEM((_CAP, _G), jnp.int32),
        pltpu.VMEM((_CAP, _G), jnp.float32),
        pltpu.VMEM((_CAP, _G), jnp.int32),
        pltpu.VMEM((_NSLOT, _G, _D), jnp.float32),
        pltpu.VMEM_SHARED((_CHUNK, _D), jnp.float32),
        pltpu.SemaphoreType.DMA((_NSLOT,)),
        pltpu.SemaphoreType.DMA((_NSLOT,)),
        pltpu.SemaphoreType.DMA,
    ],
)


def kernel(spot_x, A_rows, A_cols, A_vals):
  rows = A_rows.astype(jnp.int32)
  cols = A_cols.astype(jnp.int32)
  return _kern(spot_x, rows, cols, A_vals)


# speedup vs baseline: 3.7447x; 1.4274x over previous
import jax
import jax.numpy as jnp
from jax import lax
from jax.experimental import pallas as pl
from jax.experimental.pallas import tpu as pltpu
from jax.experimental.pallas import tpu_sc as plsc

_N = 65536
_D = 64
_NNZ = 4194304

_NS = 16
_NUM_CHUNKS = 4
_CHUNK = _N // _NUM_CHUNKS
_B = 2048
_G = 128
_CAP = 32
_NSLOT = 4
_EPT = _NNZ // _NS
_NBLK = _EPT // _B
_ROWS_PER_TILE = _CHUNK // _NS


def _sc_body(spot_hbm, rows_hbm, cols_hbm, vals_hbm, out_hbm,
             rbuf, cbuf, vbuf, ccomp, vcomp, rcomp, gbuf, accum,
             gsem, csem, ssem):
  cid = lax.axis_index("c")
  sid = lax.axis_index("s")
  iota = lax.iota(jnp.int32, 16)

  def _stage_issue(blk, par):
    base = sid * _EPT + blk * _B
    pltpu.async_copy(rows_hbm.at[pl.ds(base, _B)], rbuf.at[par], ssem)
    pltpu.async_copy(cols_hbm.at[pl.ds(base, _B)], cbuf.at[par], ssem)
    pltpu.async_copy(vals_hbm.at[pl.ds(base, _B)], vbuf.at[par], ssem)

  def _stage_wait(par):
    pltpu.make_async_copy(rows_hbm.at[pl.ds(0, _B)], rbuf.at[par], ssem).wait()
    pltpu.make_async_copy(cols_hbm.at[pl.ds(0, _B)], cbuf.at[par], ssem).wait()
    pltpu.make_async_copy(vals_hbm.at[pl.ds(0, _B)], vbuf.at[par], ssem).wait()

  def _gather_issue(j):
    @pl.when(j >= _NSLOT)
    def _():
      pltpu.make_async_copy(
          gbuf.at[j & (_NSLOT - 1)],
          accum.at[rcomp.at[(j - _NSLOT) & (_CAP - 1)]],
          csem.at[j & (_NSLOT - 1)]).wait()
    pltpu.async_copy(spot_hbm.at[ccomp.at[j & (_CAP - 1)]],
                     gbuf.at[j & (_NSLOT - 1)], gsem.at[j & (_NSLOT - 1)])

  def _gather_wait(j):
    pltpu.make_async_copy(spot_hbm.at[ccomp.at[j & (_CAP - 1)]],
                          gbuf.at[j & (_NSLOT - 1)],
                          gsem.at[j & (_NSLOT - 1)]).wait()

  def _scatter_issue(j):
    pltpu.async_copy(gbuf.at[j & (_NSLOT - 1)],
                     accum.at[rcomp.at[j & (_CAP - 1)]],
                     csem.at[j & (_NSLOT - 1)], add=True)

  def _scatter_drain(t):
    pltpu.make_async_copy(gbuf.at[t & (_NSLOT - 1)],
                          accum.at[rcomp.at[t & (_CAP - 1)]],
                          csem.at[t & (_NSLOT - 1)]).wait()

  def _bcast_lane(vec, l):
    idx = jnp.full((16, 1), l, jnp.int32)
    return lax.gather(
        vec, idx,
        lax.GatherDimensionNumbers(offset_dims=(), collapsed_slice_dims=(0,),
                                   start_index_map=(0,)),
        (1,), mode=lax.GatherScatterMode.PROMISE_IN_BOUNDS)

  def _mul(j):
    par = j & (_NSLOT - 1)
    row = j & (_CAP - 1)

    @plsc.parallel_loop(0, _G // 16, unroll=4)
    def _q(q):
      vv = vcomp[row, pl.ds(q * 16, 16)]
      e0 = q * 16
      for l in range(16):
        bc = _bcast_lane(vv, l)
        for k in range(_D // 16):
          gbuf[par, e0 + l, pl.ds(k * 16, 16)] = (
              gbuf[par, e0 + l, pl.ds(k * 16, 16)] * bc)

  def _idxinit(r, carry):
    for k in range(_G // 16):
      ccomp[r, pl.ds(k * 16, 16)] = jnp.zeros((16,), jnp.int32)
      rcomp[r, pl.ds(k * 16, 16)] = jnp.zeros((16,), jnp.int32)
    return carry
  lax.fori_loop(0, _CAP, _idxinit, 0)

  for p in range(_NUM_CHUNKS // 2):
    chunk = 2 * cid + p
    lo = chunk * _CHUNK

    def _zinit(e, carry):
      for k in range(_D // 16):
        gbuf[0, e, pl.ds(k * 16, 16)] = jnp.zeros((16,), jnp.float32)
      return carry
    lax.fori_loop(0, _G, _zinit, 0)
    for z in range(_ROWS_PER_TILE // _G):
      pltpu.sync_copy(gbuf.at[0],
                      accum.at[pl.ds(sid * _ROWS_PER_TILE + z * _G, _G)])
    plsc.subcore_barrier()

    _stage_issue(0, 0)

    def _block(blk, carry):
      cvec0, proc0 = carry
      done0 = lax.shift_right_logical(cvec0[0], 7)
      pb = blk & 1
      _stage_wait(pb)

      @pl.when(blk + 1 < _NBLK)
      def _():
        _stage_issue(blk + 1, 1 - pb)

      def _compact(i, cvec):
        r = rbuf[pb, pl.ds(i * 16, 16)]
        c = cbuf[pb, pl.ds(i * 16, 16)]
        v = vbuf[pb, pl.ds(i * 16, 16)]
        rl = r - lo
        m = (rl >= 0) & (rl < _CHUNK)
        inc = jnp.cumsum(jnp.where(m, jnp.int32(1), jnp.int32(0)))
        pos = cvec + inc - 1
        pj = lax.bitwise_and(lax.shift_right_logical(pos, 7),
                             jnp.int32(_CAP - 1))
        pi = lax.bitwise_and(pos, jnp.int32(_G - 1))
        plsc.store_scatter(ccomp, [pj, pi], c, mask=m)
        plsc.store_scatter(vcomp, [pj, pi], v, mask=m)
        plsc.store_scatter(rcomp, [pj, pi], rl, mask=m)
        return _bcast_lane(cvec + inc, 15)

      cvec1 = plsc.parallel_loop(0, _B // 16, unroll=4, carry=cvec0)(_compact)
      done1 = lax.shift_right_logical(cvec1[0], 7)

      def _top(t, c2):
        _gather_issue(t)
        return c2
      lax.fori_loop(jnp.minimum(proc0 + 2, done0),
                    jnp.minimum(proc0 + 2, done1), _top, 0)

      proc1 = jnp.maximum(done1 - 2, proc0)

      def _chunkproc(j, c2):
        _gather_issue(j + 2)
        _gather_wait(j)
        _mul(j)
        _scatter_issue(j)
        return c2
      lax.fori_loop(proc0, proc1, _chunkproc, 0)
      return (cvec1, proc1)

    cvec, proc = lax.fori_loop(
        0, _NBLK, _block, (jnp.zeros((16,), jnp.int32), jnp.int32(0)))
    count = cvec[0]
    done = lax.shift_right_logical(count, 7)
    rem = lax.bitwise_and(count, jnp.int32(_G - 1))

    def _left(j, c2):
      @pl.when(j + 2 < done)
      def _():
        _gather_issue(j + 2)
      _gather_wait(j)
      _mul(j)
      _scatter_issue(j)
      return c2
    lax.fori_loop(proc, done, _left, 0)

    @pl.when(rem > 0)
    def _():
      def _pad(g, carry):
        row = lax.bitwise_and(lax.shift_right_logical(g, 3),
                              jnp.int32(_CAP - 1))
        col = lax.bitwise_and(g, jnp.int32(7)) * 16
        old = vcomp[row, pl.ds(col, 16)]
        keep = (g * 16 + iota) < count
        vcomp[row, pl.ds(col, 16)] = jnp.where(keep, old, jnp.float32(0.0))
        return carry
      lax.fori_loop(lax.shift_right_logical(count, 4), (done + 1) * 8,
                    _pad, 0)
      _gather_issue(done)
      _gather_wait(done)
      _mul(done)
      _scatter_issue(done)

    total = done + jnp.where(rem > 0, jnp.int32(1), jnp.int32(0))

    def _fin(t, c2):
      _scatter_drain(t)
      return c2
    lax.fori_loop(jnp.maximum(total - _NSLOT, 0), total, _fin, 0)

    plsc.subcore_barrier()

    pltpu.sync_copy(
        accum.at[pl.ds(sid * _ROWS_PER_TILE, _ROWS_PER_TILE)],
        out_hbm.at[pl.ds(lo + sid * _ROWS_PER_TILE, _ROWS_PER_TILE)])
    plsc.subcore_barrier()


_kern = pl.kernel(
    _sc_body,
    out_type=jax.ShapeDtypeStruct((_N, _D), jnp.float32),
    mesh=plsc.VectorSubcoreMesh(core_axis_name="c", subcore_axis_name="s"),
    compiler_params=pltpu.CompilerParams(
        needs_layout_passes=False, use_tc_tiling_on_sc=False),
    scratch_types=[
        pltpu.VMEM((2, _B), jnp.int32),
        pltpu.VMEM((2, _B), jnp.int32),
        pltpu.VMEM((2, _B), jnp.float32),
        pltpu.VMEM((_CAP, _G), jnp.int32),
        pltpu.VMEM((_CAP, _G), jnp.float32),
        pltpu.VMEM((_CAP, _G), jnp.int32),
        pltpu.VMEM((_NSLOT, _G, _D), jnp.float32),
        pltpu.VMEM_SHARED((_CHUNK, _D), jnp.float32),
        pltpu.SemaphoreType.DMA((_NSLOT,)),
        pltpu.SemaphoreType.DMA((_NSLOT,)),
        pltpu.SemaphoreType.DMA,
    ],
)


def kernel(spot_x, A_rows, A_cols, A_vals):
  rows = A_rows.astype(jnp.int32)
  cols = A_cols.astype(jnp.int32)
  return _kern(spot_x, rows, cols, A_vals)


# speedup vs baseline: 3.9932x; 1.0663x over previous
import jax
import jax.numpy as jnp
from jax import lax
from jax.experimental import pallas as pl
from jax.experimental.pallas import tpu as pltpu
from jax.experimental.pallas import tpu_sc as plsc

_N = 65536
_D = 64
_NNZ = 4194304

_NS = 16
_NUM_CHUNKS = 4
_CHUNK = _N // _NUM_CHUNKS
_B = 2048
_G = 128
_CAP = 32
_NSLOT = 4
_EPT = _NNZ // _NS
_NBLK = _EPT // _B
_ROWS_PER_TILE = _CHUNK // _NS


def _sc_body(spot_hbm, rows_hbm, cols_hbm, vals_hbm, out_hbm,
             rbuf, cbuf, vbuf, ccomp, vcomp, rcomp, gbuf, accum,
             gsem, csem, ssem):
  cid = lax.axis_index("c")
  sid = lax.axis_index("s")
  iota = lax.iota(jnp.int32, 16)

  def _stage_issue(blk, par):
    base = sid * _EPT + blk * _B
    pltpu.async_copy(rows_hbm.at[pl.ds(base, _B)], rbuf.at[par], ssem)
    pltpu.async_copy(cols_hbm.at[pl.ds(base, _B)], cbuf.at[par], ssem)
    pltpu.async_copy(vals_hbm.at[pl.ds(base, _B)], vbuf.at[par], ssem)

  def _stage_wait(par):
    pltpu.make_async_copy(rows_hbm.at[pl.ds(0, _B)], rbuf.at[par], ssem).wait()
    pltpu.make_async_copy(cols_hbm.at[pl.ds(0, _B)], cbuf.at[par], ssem).wait()
    pltpu.make_async_copy(vals_hbm.at[pl.ds(0, _B)], vbuf.at[par], ssem).wait()

  def _gather_issue(j):
    @pl.when(j >= _NSLOT)
    def _():
      pltpu.make_async_copy(
          gbuf.at[j & (_NSLOT - 1)],
          accum.at[rcomp.at[(j - _NSLOT) & (_CAP - 1)]],
          csem.at[j & (_NSLOT - 1)]).wait()
    pltpu.async_copy(spot_hbm.at[ccomp.at[j & (_CAP - 1)]],
                     gbuf.at[j & (_NSLOT - 1)], gsem.at[j & (_NSLOT - 1)])

  def _gather_wait(j):
    pltpu.make_async_copy(spot_hbm.at[ccomp.at[j & (_CAP - 1)]],
                          gbuf.at[j & (_NSLOT - 1)],
                          gsem.at[j & (_NSLOT - 1)]).wait()

  def _scatter_issue(j):
    pltpu.async_copy(gbuf.at[j & (_NSLOT - 1)],
                     accum.at[rcomp.at[j & (_CAP - 1)]],
                     csem.at[j & (_NSLOT - 1)], add=True)

  def _scatter_drain(t):
    pltpu.make_async_copy(gbuf.at[t & (_NSLOT - 1)],
                          accum.at[rcomp.at[t & (_CAP - 1)]],
                          csem.at[t & (_NSLOT - 1)]).wait()

  def _bcast_lane(vec, l):
    idx = jnp.full((16, 1), l, jnp.int32)
    return lax.gather(
        vec, idx,
        lax.GatherDimensionNumbers(offset_dims=(), collapsed_slice_dims=(0,),
                                   start_index_map=(0,)),
        (1,), mode=lax.GatherScatterMode.PROMISE_IN_BOUNDS)

  def _mul(j):
    par = j & (_NSLOT - 1)
    row = j & (_CAP - 1)

    @plsc.parallel_loop(0, _G // 16, unroll=8)
    def _q(q):
      vv = vcomp[row, pl.ds(q * 16, 16)]
      e0 = q * 16
      for l in range(16):
        bc = _bcast_lane(vv, l)
        for k in range(_D // 16):
          gbuf[par, e0 + l, pl.ds(k * 16, 16)] = (
              gbuf[par, e0 + l, pl.ds(k * 16, 16)] * bc)

  def _idxinit(r, carry):
    for k in range(_G // 16):
      ccomp[r, pl.ds(k * 16, 16)] = jnp.zeros((16,), jnp.int32)
      rcomp[r, pl.ds(k * 16, 16)] = jnp.zeros((16,), jnp.int32)
    return carry
  lax.fori_loop(0, _CAP, _idxinit, 0)

  for p in range(_NUM_CHUNKS // 2):
    chunk = 2 * cid + p
    lo = chunk * _CHUNK

    def _zinit(e, carry):
      for k in range(_D // 16):
        gbuf[0, e, pl.ds(k * 16, 16)] = jnp.zeros((16,), jnp.float32)
      return carry
    lax.fori_loop(0, _G, _zinit, 0)
    for z in range(_ROWS_PER_TILE // _G):
      pltpu.sync_copy(gbuf.at[0],
                      accum.at[pl.ds(sid * _ROWS_PER_TILE + z * _G, _G)])
    plsc.subcore_barrier()

    _stage_issue(0, 0)

    def _block(blk, carry):
      cvec0, proc0 = carry
      done0 = lax.shift_right_logical(cvec0[0], 7)
      pb = blk & 1
      _stage_wait(pb)

      @pl.when(blk + 1 < _NBLK)
      def _():
        _stage_issue(blk + 1, 1 - pb)

      def _compact(i, cvec):
        r = rbuf[pb, pl.ds(i * 16, 16)]
        c = cbuf[pb, pl.ds(i * 16, 16)]
        v = vbuf[pb, pl.ds(i * 16, 16)]
        rl = r - lo
        m = (rl >= 0) & (rl < _CHUNK)
        inc = jnp.cumsum(jnp.where(m, jnp.int32(1), jnp.int32(0)))
        pos = cvec + inc - 1
        pj = lax.bitwise_and(lax.shift_right_logical(pos, 7),
                             jnp.int32(_CAP - 1))
        pi = lax.bitwise_and(pos, jnp.int32(_G - 1))
        plsc.store_scatter(ccomp, [pj, pi], c, mask=m)
        plsc.store_scatter(vcomp, [pj, pi], v, mask=m)
        plsc.store_scatter(rcomp, [pj, pi], rl, mask=m)
        return _bcast_lane(cvec + inc, 15)

      cvec1 = plsc.parallel_loop(0, _B // 16, unroll=4, carry=cvec0)(_compact)
      done1 = lax.shift_right_logical(cvec1[0], 7)

      def _top(t, c2):
        _gather_issue(t)
        return c2
      lax.fori_loop(jnp.minimum(proc0 + 2, done0),
                    jnp.minimum(proc0 + 2, done1), _top, 0)

      proc1 = jnp.maximum(done1 - 2, proc0)

      def _chunkproc(j, c2):
        _gather_issue(j + 2)
        _gather_wait(j)
        _mul(j)
        _scatter_issue(j)
        return c2
      lax.fori_loop(proc0, proc1, _chunkproc, 0)
      return (cvec1, proc1)

    cvec, proc = lax.fori_loop(
        0, _NBLK, _block, (jnp.zeros((16,), jnp.int32), jnp.int32(0)))
    count = cvec[0]
    done = lax.shift_right_logical(count, 7)
    rem = lax.bitwise_and(count, jnp.int32(_G - 1))

    def _left(j, c2):
      @pl.when(j + 2 < done)
      def _():
        _gather_issue(j + 2)
      _gather_wait(j)
      _mul(j)
      _scatter_issue(j)
      return c2
    lax.fori_loop(proc, done, _left, 0)

    @pl.when(rem > 0)
    def _():
      def _pad(g, carry):
        row = lax.bitwise_and(lax.shift_right_logical(g, 3),
                              jnp.int32(_CAP - 1))
        col = lax.bitwise_and(g, jnp.int32(7)) * 16
        old = vcomp[row, pl.ds(col, 16)]
        keep = (g * 16 + iota) < count
        vcomp[row, pl.ds(col, 16)] = jnp.where(keep, old, jnp.float32(0.0))
        return carry
      lax.fori_loop(lax.shift_right_logical(count, 4), (done + 1) * 8,
                    _pad, 0)
      _gather_issue(done)
      _gather_wait(done)
      _mul(done)
      _scatter_issue(done)

    total = done + jnp.where(rem > 0, jnp.int32(1), jnp.int32(0))

    def _fin(t, c2):
      _scatter_drain(t)
      return c2
    lax.fori_loop(jnp.maximum(total - _NSLOT, 0), total, _fin, 0)

    plsc.subcore_barrier()

    pltpu.sync_copy(
        accum.at[pl.ds(sid * _ROWS_PER_TILE, _ROWS_PER_TILE)],
        out_hbm.at[pl.ds(lo + sid * _ROWS_PER_TILE, _ROWS_PER_TILE)])
    plsc.subcore_barrier()


_kern = pl.kernel(
    _sc_body,
    out_type=jax.ShapeDtypeStruct((_N, _D), jnp.float32),
    mesh=plsc.VectorSubcoreMesh(core_axis_name="c", subcore_axis_name="s"),
    compiler_params=pltpu.CompilerParams(
        needs_layout_passes=False, use_tc_tiling_on_sc=False),
    scratch_types=[
        pltpu.VMEM((2, _B), jnp.int32),
        pltpu.VMEM((2, _B), jnp.int32),
        pltpu.VMEM((2, _B), jnp.float32),
        pltpu.VMEM((_CAP, _G), jnp.int32),
        pltpu.VMEM((_CAP, _G), jnp.float32),
        pltpu.VMEM((_CAP, _G), jnp.int32),
        pltpu.VMEM((_NSLOT, _G, _D), jnp.float32),
        pltpu.VMEM_SHARED((_CHUNK, _D), jnp.float32),
        pltpu.SemaphoreType.DMA((_NSLOT,)),
        pltpu.SemaphoreType.DMA((_NSLOT,)),
        pltpu.SemaphoreType.DMA,
    ],
)


def kernel(spot_x, A_rows, A_cols, A_vals):
  rows = A_rows.astype(jnp.int32)
  cols = A_cols.astype(jnp.int32)
  return _kern(spot_x, rows, cols, A_vals)


# speedup vs baseline: 3.9968x; 1.0009x over previous
import jax
import jax.numpy as jnp
from jax import lax
from jax.experimental import pallas as pl
from jax.experimental.pallas import tpu as pltpu
from jax.experimental.pallas import tpu_sc as plsc

_N = 65536
_D = 64
_NNZ = 4194304

_NS = 16
_NUM_CHUNKS = 4
_CHUNK = _N // _NUM_CHUNKS
_B = 2048
_G = 128
_CAP = 32
_NSLOT = 4
_EPT = _NNZ // _NS
_NBLK = _EPT // _B
_ROWS_PER_TILE = _CHUNK // _NS


def _sc_body(spot_hbm, rows_hbm, cols_hbm, vals_hbm, out_hbm,
             rbuf, cbuf, vbuf, ccomp, vcomp, rcomp, gbuf, accum,
             gsem, csem, ssem):
  cid = lax.axis_index("c")
  sid = lax.axis_index("s")
  iota = lax.iota(jnp.int32, 16)

  def _stage_issue(blk, par):
    base = sid * _EPT + blk * _B
    pltpu.async_copy(rows_hbm.at[pl.ds(base, _B)], rbuf.at[par], ssem)
    pltpu.async_copy(cols_hbm.at[pl.ds(base, _B)], cbuf.at[par], ssem)
    pltpu.async_copy(vals_hbm.at[pl.ds(base, _B)], vbuf.at[par], ssem)

  def _stage_wait(par):
    pltpu.make_async_copy(rows_hbm.at[pl.ds(0, _B)], rbuf.at[par], ssem).wait()
    pltpu.make_async_copy(cols_hbm.at[pl.ds(0, _B)], cbuf.at[par], ssem).wait()
    pltpu.make_async_copy(vals_hbm.at[pl.ds(0, _B)], vbuf.at[par], ssem).wait()

  def _gather_issue(j):
    @pl.when(j >= _NSLOT)
    def _():
      pltpu.make_async_copy(
          gbuf.at[j & (_NSLOT - 1)],
          accum.at[rcomp.at[(j - _NSLOT) & (_CAP - 1)]],
          csem.at[j & (_NSLOT - 1)]).wait()
    pltpu.async_copy(spot_hbm.at[ccomp.at[j & (_CAP - 1)]],
                     gbuf.at[j & (_NSLOT - 1)], gsem.at[j & (_NSLOT - 1)])

  def _gather_wait(j):
    pltpu.make_async_copy(spot_hbm.at[ccomp.at[j & (_CAP - 1)]],
                          gbuf.at[j & (_NSLOT - 1)],
                          gsem.at[j & (_NSLOT - 1)]).wait()

  def _scatter_issue(j):
    pltpu.async_copy(gbuf.at[j & (_NSLOT - 1)],
                     accum.at[rcomp.at[j & (_CAP - 1)]],
                     csem.at[j & (_NSLOT - 1)], add=True)

  def _scatter_drain(t):
    pltpu.make_async_copy(gbuf.at[t & (_NSLOT - 1)],
                          accum.at[rcomp.at[t & (_CAP - 1)]],
                          csem.at[t & (_NSLOT - 1)]).wait()

  def _bcast_lane(vec, l):
    idx = jnp.full((16, 1), l, jnp.int32)
    return lax.gather(
        vec, idx,
        lax.GatherDimensionNumbers(offset_dims=(), collapsed_slice_dims=(0,),
                                   start_index_map=(0,)),
        (1,), mode=lax.GatherScatterMode.PROMISE_IN_BOUNDS)

  def _mul(j):
    par = j & (_NSLOT - 1)
    row = j & (_CAP - 1)

    @plsc.parallel_loop(0, _G // 16, unroll=8)
    def _q(q):
      vv = vcomp[row, pl.ds(q * 16, 16)]
      e0 = q * 16
      for l in range(16):
        bc = _bcast_lane(vv, l)
        for k in range(_D // 16):
          gbuf[par, e0 + l, pl.ds(k * 16, 16)] = (
              gbuf[par, e0 + l, pl.ds(k * 16, 16)] * bc)

  def _idxinit(r, carry):
    for k in range(_G // 16):
      ccomp[r, pl.ds(k * 16, 16)] = jnp.zeros((16,), jnp.int32)
      rcomp[r, pl.ds(k * 16, 16)] = jnp.zeros((16,), jnp.int32)
    return carry
  lax.fori_loop(0, _CAP, _idxinit, 0)

  for p in range(_NUM_CHUNKS // 2):
    chunk = 2 * cid + p
    lo = chunk * _CHUNK

    def _zinit(e, carry):
      for k in range(_D // 16):
        gbuf[0, e, pl.ds(k * 16, 16)] = jnp.zeros((16,), jnp.float32)
      return carry
    lax.fori_loop(0, _G, _zinit, 0)
    for z in range(_ROWS_PER_TILE // _G):
      pltpu.sync_copy(gbuf.at[0],
                      accum.at[pl.ds(sid * _ROWS_PER_TILE + z * _G, _G)])
    plsc.subcore_barrier()

    _stage_issue(0, 0)

    def _block(blk, carry):
      cvec0, proc0 = carry
      done0 = lax.shift_right_logical(cvec0[0], 7)
      pb = blk & 1
      _stage_wait(pb)

      @pl.when(blk + 1 < _NBLK)
      def _():
        _stage_issue(blk + 1, 1 - pb)

      def _compact(i, cvec):
        r = rbuf[pb, pl.ds(i * 16, 16)]
        c = cbuf[pb, pl.ds(i * 16, 16)]
        v = vbuf[pb, pl.ds(i * 16, 16)]
        rl = r - lo
        m = (rl >= 0) & (rl < _CHUNK)
        inc = jnp.cumsum(jnp.where(m, jnp.int32(1), jnp.int32(0)))
        pos = cvec + inc - 1
        pj = lax.bitwise_and(lax.shift_right_logical(pos, 7),
                             jnp.int32(_CAP - 1))
        pi = lax.bitwise_and(pos, jnp.int32(_G - 1))
        plsc.store_scatter(ccomp, [pj, pi], c, mask=m)
        plsc.store_scatter(vcomp, [pj, pi], v, mask=m)
        plsc.store_scatter(rcomp, [pj, pi], rl, mask=m)
        return _bcast_lane(cvec + inc, 15)

      cvec1 = plsc.parallel_loop(0, _B // 16, unroll=8, carry=cvec0)(_compact)
      done1 = lax.shift_right_logical(cvec1[0], 7)

      def _top(t, c2):
        _gather_issue(t)
        return c2
      lax.fori_loop(jnp.minimum(proc0 + 2, done0),
                    jnp.minimum(proc0 + 2, done1), _top, 0)

      proc1 = jnp.maximum(done1 - 2, proc0)

      def _chunkproc(j, c2):
        _gather_issue(j + 2)
        _gather_wait(j)
        _mul(j)
        _scatter_issue(j)
        return c2
      lax.fori_loop(proc0, proc1, _chunkproc, 0)
      return (cvec1, proc1)

    cvec, proc = lax.fori_loop(
        0, _NBLK, _block, (jnp.zeros((16,), jnp.int32), jnp.int32(0)))
    count = cvec[0]
    done = lax.shift_right_logical(count, 7)
    rem = lax.bitwise_and(count, jnp.int32(_G - 1))

    def _left(j, c2):
      @pl.when(j + 2 < done)
      def _():
        _gather_issue(j + 2)
      _gather_wait(j)
      _mul(j)
      _scatter_issue(j)
      return c2
    lax.fori_loop(proc, done, _left, 0)

    @pl.when(rem > 0)
    def _():
      def _pad(g, carry):
        row = lax.bitwise_and(lax.shift_right_logical(g, 3),
                              jnp.int32(_CAP - 1))
        col = lax.bitwise_and(g, jnp.int32(7)) * 16
        old = vcomp[row, pl.ds(col, 16)]
        keep = (g * 16 + iota) < count
        vcomp[row, pl.ds(col, 16)] = jnp.where(keep, old, jnp.float32(0.0))
        return carry
      lax.fori_loop(lax.shift_right_logical(count, 4), (done + 1) * 8,
                    _pad, 0)
      _gather_issue(done)
      _gather_wait(done)
      _mul(done)
      _scatter_issue(done)

    total = done + jnp.where(rem > 0, jnp.int32(1), jnp.int32(0))

    def _fin(t, c2):
      _scatter_drain(t)
      return c2
    lax.fori_loop(jnp.maximum(total - _NSLOT, 0), total, _fin, 0)

    plsc.subcore_barrier()

    pltpu.sync_copy(
        accum.at[pl.ds(sid * _ROWS_PER_TILE, _ROWS_PER_TILE)],
        out_hbm.at[pl.ds(lo + sid * _ROWS_PER_TILE, _ROWS_PER_TILE)])
    plsc.subcore_barrier()


_kern = pl.kernel(
    _sc_body,
    out_type=jax.ShapeDtypeStruct((_N, _D), jnp.float32),
    mesh=plsc.VectorSubcoreMesh(core_axis_name="c", subcore_axis_name="s"),
    compiler_params=pltpu.CompilerParams(
        needs_layout_passes=False, use_tc_tiling_on_sc=False),
    scratch_types=[
        pltpu.VMEM((2, _B), jnp.int32),
        pltpu.VMEM((2, _B), jnp.int32),
        pltpu.VMEM((2, _B), jnp.float32),
        pltpu.VMEM((_CAP, _G), jnp.int32),
        pltpu.VMEM((_CAP, _G), jnp.float32),
        pltpu.VMEM((_CAP, _G), jnp.int32),
        pltpu.VMEM((_NSLOT, _G, _D), jnp.float32),
        pltpu.VMEM_SHARED((_CHUNK, _D), jnp.float32),
        pltpu.SemaphoreType.DMA((_NSLOT,)),
        pltpu.SemaphoreType.DMA((_NSLOT,)),
        pltpu.SemaphoreType.DMA,
    ],
)


def kernel(spot_x, A_rows, A_cols, A_vals):
  rows = A_rows.astype(jnp.int32)
  cols = A_cols.astype(jnp.int32)
  return _kern(spot_x, rows, cols, A_vals)


# speedup vs baseline: 4.1007x; 1.0260x over previous
import jax
import jax.numpy as jnp
from jax import lax
from jax.experimental import pallas as pl
from jax.experimental.pallas import tpu as pltpu
from jax.experimental.pallas import tpu_sc as plsc

_N = 65536
_D = 64
_NNZ = 4194304

_NS = 16
_NUM_CHUNKS = 4
_CHUNK = _N // _NUM_CHUNKS
_B = 1024
_G = 128
_CAP = 32
_NSLOT = 4
_EPT = _NNZ // _NS
_NBLK = _EPT // _B
_ROWS_PER_TILE = _CHUNK // _NS


def _sc_body(spot_hbm, rows_hbm, cols_hbm, vals_hbm, out_hbm,
             rbuf, cbuf, vbuf, ccomp, vcomp, rcomp, gbuf, accum,
             gsem, csem, ssem):
  cid = lax.axis_index("c")
  sid = lax.axis_index("s")
  iota = lax.iota(jnp.int32, 16)

  def _stage_issue(blk, par):
    base = sid * _EPT + blk * _B
    pltpu.async_copy(rows_hbm.at[pl.ds(base, _B)], rbuf.at[par], ssem)
    pltpu.async_copy(cols_hbm.at[pl.ds(base, _B)], cbuf.at[par], ssem)
    pltpu.async_copy(vals_hbm.at[pl.ds(base, _B)], vbuf.at[par], ssem)

  def _stage_wait(par):
    pltpu.make_async_copy(rows_hbm.at[pl.ds(0, _B)], rbuf.at[par], ssem).wait()
    pltpu.make_async_copy(cols_hbm.at[pl.ds(0, _B)], cbuf.at[par], ssem).wait()
    pltpu.make_async_copy(vals_hbm.at[pl.ds(0, _B)], vbuf.at[par], ssem).wait()

  def _gather_issue(j):
    @pl.when(j >= _NSLOT)
    def _():
      pltpu.make_async_copy(
          gbuf.at[j & (_NSLOT - 1)],
          accum.at[rcomp.at[(j - _NSLOT) & (_CAP - 1)]],
          csem.at[j & (_NSLOT - 1)]).wait()
    pltpu.async_copy(spot_hbm.at[ccomp.at[j & (_CAP - 1)]],
                     gbuf.at[j & (_NSLOT - 1)], gsem.at[j & (_NSLOT - 1)])

  def _gather_wait(j):
    pltpu.make_async_copy(spot_hbm.at[ccomp.at[j & (_CAP - 1)]],
                          gbuf.at[j & (_NSLOT - 1)],
                          gsem.at[j & (_NSLOT - 1)]).wait()

  def _scatter_issue(j):
    pltpu.async_copy(gbuf.at[j & (_NSLOT - 1)],
                     accum.at[rcomp.at[j & (_CAP - 1)]],
                     csem.at[j & (_NSLOT - 1)], add=True)

  def _scatter_drain(t):
    pltpu.make_async_copy(gbuf.at[t & (_NSLOT - 1)],
                          accum.at[rcomp.at[t & (_CAP - 1)]],
                          csem.at[t & (_NSLOT - 1)]).wait()

  def _bcast_lane(vec, l):
    idx = jnp.full((16, 1), l, jnp.int32)
    return lax.gather(
        vec, idx,
        lax.GatherDimensionNumbers(offset_dims=(), collapsed_slice_dims=(0,),
                                   start_index_map=(0,)),
        (1,), mode=lax.GatherScatterMode.PROMISE_IN_BOUNDS)

  def _mul(j):
    par = j & (_NSLOT - 1)
    row = j & (_CAP - 1)

    @plsc.parallel_loop(0, _G // 16, unroll=8)
    def _q(q):
      vv = vcomp[row, pl.ds(q * 16, 16)]
      e0 = q * 16
      for l in range(16):
        bc = _bcast_lane(vv, l)
        for k in range(_D // 16):
          gbuf[par, e0 + l, pl.ds(k * 16, 16)] = (
              gbuf[par, e0 + l, pl.ds(k * 16, 16)] * bc)

  def _idxinit(r, carry):
    for k in range(_G // 16):
      ccomp[r, pl.ds(k * 16, 16)] = jnp.zeros((16,), jnp.int32)
      rcomp[r, pl.ds(k * 16, 16)] = jnp.zeros((16,), jnp.int32)
    return carry
  lax.fori_loop(0, _CAP, _idxinit, 0)

  for p in range(_NUM_CHUNKS // 2):
    chunk = 2 * cid + p
    lo = chunk * _CHUNK

    def _zinit(e, carry):
      for k in range(_D // 16):
        gbuf[0, e, pl.ds(k * 16, 16)] = jnp.zeros((16,), jnp.float32)
      return carry
    lax.fori_loop(0, _G, _zinit, 0)
    for z in range(_ROWS_PER_TILE // _G):
      pltpu.sync_copy(gbuf.at[0],
                      accum.at[pl.ds(sid * _ROWS_PER_TILE + z * _G, _G)])
    plsc.subcore_barrier()

    _stage_issue(0, 0)

    def _block(blk, carry):
      cvec0, proc0 = carry
      done0 = lax.shift_right_logical(cvec0[0], 7)
      pb = blk & 1
      _stage_wait(pb)

      @pl.when(blk + 1 < _NBLK)
      def _():
        _stage_issue(blk + 1, 1 - pb)

      def _compact(i, cvec):
        r = rbuf[pb, pl.ds(i * 16, 16)]
        c = cbuf[pb, pl.ds(i * 16, 16)]
        v = vbuf[pb, pl.ds(i * 16, 16)]
        rl = r - lo
        m = (rl >= 0) & (rl < _CHUNK)
        inc = jnp.cumsum(jnp.where(m, jnp.int32(1), jnp.int32(0)))
        pos = cvec + inc - 1
        pj = lax.bitwise_and(lax.shift_right_logical(pos, 7),
                             jnp.int32(_CAP - 1))
        pi = lax.bitwise_and(pos, jnp.int32(_G - 1))
        plsc.store_scatter(ccomp, [pj, pi], c, mask=m)
        plsc.store_scatter(vcomp, [pj, pi], v, mask=m)
        plsc.store_scatter(rcomp, [pj, pi], rl, mask=m)
        return _bcast_lane(cvec + inc, 15)

      cvec1 = plsc.parallel_loop(0, _B // 16, unroll=8, carry=cvec0)(_compact)
      done1 = lax.shift_right_logical(cvec1[0], 7)

      def _top(t, c2):
        _gather_issue(t)
        return c2
      lax.fori_loop(jnp.minimum(proc0 + 2, done0),
                    jnp.minimum(proc0 + 2, done1), _top, 0)

      proc1 = jnp.maximum(done1 - 2, proc0)

      def _chunkproc(j, c2):
        _gather_issue(j + 2)
        _gather_wait(j)
        _mul(j)
        _scatter_issue(j)
        return c2
      lax.fori_loop(proc0, proc1, _chunkproc, 0)
      return (cvec1, proc1)

    cvec, proc = lax.fori_loop(
        0, _NBLK, _block, (jnp.zeros((16,), jnp.int32), jnp.int32(0)))
    count = cvec[0]
    done = lax.shift_right_logical(count, 7)
    rem = lax.bitwise_and(count, jnp.int32(_G - 1))

    def _left(j, c2):
      @pl.when(j + 2 < done)
      def _():
        _gather_issue(j + 2)
      _gather_wait(j)
      _mul(j)
      _scatter_issue(j)
      return c2
    lax.fori_loop(proc, done, _left, 0)

    @pl.when(rem > 0)
    def _():
      def _pad(g, carry):
        row = lax.bitwise_and(lax.shift_right_logical(g, 3),
                              jnp.int32(_CAP - 1))
        col = lax.bitwise_and(g, jnp.int32(7)) * 16
        old = vcomp[row, pl.ds(col, 16)]
        keep = (g * 16 + iota) < count
        vcomp[row, pl.ds(col, 16)] = jnp.where(keep, old, jnp.float32(0.0))
        return carry
      lax.fori_loop(lax.shift_right_logical(count, 4), (done + 1) * 8,
                    _pad, 0)
      _gather_issue(done)
      _gather_wait(done)
      _mul(done)
      _scatter_issue(done)

    total = done + jnp.where(rem > 0, jnp.int32(1), jnp.int32(0))

    def _fin(t, c2):
      _scatter_drain(t)
      return c2
    lax.fori_loop(jnp.maximum(total - _NSLOT, 0), total, _fin, 0)

    plsc.subcore_barrier()

    pltpu.sync_copy(
        accum.at[pl.ds(sid * _ROWS_PER_TILE, _ROWS_PER_TILE)],
        out_hbm.at[pl.ds(lo + sid * _ROWS_PER_TILE, _ROWS_PER_TILE)])
    plsc.subcore_barrier()


_kern = pl.kernel(
    _sc_body,
    out_type=jax.ShapeDtypeStruct((_N, _D), jnp.float32),
    mesh=plsc.VectorSubcoreMesh(core_axis_name="c", subcore_axis_name="s"),
    compiler_params=pltpu.CompilerParams(
        needs_layout_passes=False, use_tc_tiling_on_sc=False),
    scratch_types=[
        pltpu.VMEM((2, _B), jnp.int32),
        pltpu.VMEM((2, _B), jnp.int32),
        pltpu.VMEM((2, _B), jnp.float32),
        pltpu.VMEM((_CAP, _G), jnp.int32),
        pltpu.VMEM((_CAP, _G), jnp.float32),
        pltpu.VMEM((_CAP, _G), jnp.int32),
        pltpu.VMEM((_NSLOT, _G, _D), jnp.float32),
        pltpu.VMEM_SHARED((_CHUNK, _D), jnp.float32),
        pltpu.SemaphoreType.DMA((_NSLOT,)),
        pltpu.SemaphoreType.DMA((_NSLOT,)),
        pltpu.SemaphoreType.DMA,
    ],
)


def kernel(spot_x, A_rows, A_cols, A_vals):
  rows = A_rows.astype(jnp.int32)
  cols = A_cols.astype(jnp.int32)
  return _kern(spot_x, rows, cols, A_vals)
